# baseline (speedup 1.0000x reference)
import functools

import jax
import jax.numpy as jnp
from jax.experimental import pallas as pl
from jax.experimental.pallas import tpu as pltpu

_N = 8192
_D_IN = 2048
_D_OUT = 2048
_H1 = 100
_H2 = 50
_EPS = 1e-5
_CHUNK = 512
_DEPTH = 6


def _rm_dot(a, b):
    return jax.lax.dot_general(
        a, b, (((1,), (1,)), ((), ())),
        preferred_element_type=jnp.float32)


def _mlp_pipeline(x_hbm, w1_ref, b1_ref, w2_ref, b2_ref, w3_ref, b3_ref,
                  m0_ref, v0_ref, m1_ref, v1_ref, m2_ref, v2_ref, out_hbm,
                  xbuf, obuf, insems, outsems, w1s, b1s, w2s, b2s, w3s, b3s):
    nch = _N // _CHUNK

    def in_copy(c, slot):
        return pltpu.make_async_copy(
            x_hbm.at[pl.ds(c * _CHUNK, _CHUNK), :], xbuf.at[slot],
            insems.at[slot])

    def out_copy(c, slot):
        return pltpu.make_async_copy(
            obuf.at[slot], out_hbm.at[pl.ds(c * _CHUNK, _CHUNK), :],
            outsems.at[slot])

    for s in range(_DEPTH):
        in_copy(s, s).start()

    s0 = jax.lax.rsqrt(v0_ref[...] + _EPS)
    w1s[...] = (w1_ref[...] * s0).astype(jnp.bfloat16)
    b1s[...] = b1_ref[...] - _rm_dot(m0_ref[...] * s0, w1_ref[...])
    s1 = jax.lax.rsqrt(v1_ref[...] + _EPS)
    s2 = jax.lax.rsqrt(v2_ref[...] + _EPS)
    w2s[...] = (w2_ref[...] * s1).astype(jnp.bfloat16)
    b2s[...] = b2_ref[...] - _rm_dot(m1_ref[...] * s1, w2_ref[...])
    w3s[...] = (w3_ref[...] * s2).astype(jnp.bfloat16)
    b3s[...] = b3_ref[...] - _rm_dot(m2_ref[...] * s2, w3_ref[...])

    for c in range(nch):
        slot = c % _DEPTH
        in_copy(c, slot).wait()
        if c >= _DEPTH:
            out_copy(c - _DEPTH, slot).wait()
        h = jnp.maximum(_rm_dot(xbuf[slot].astype(jnp.bfloat16), w1s[...]) + b1s[...], 0.0)
        g = jnp.maximum(_rm_dot(h.astype(jnp.bfloat16), w2s[...]) + b2s[...], 0.0)
        obuf[slot] = _rm_dot(g.astype(jnp.bfloat16), w3s[...]) + b3s[...]
        out_copy(c, slot).start()
        if c + _DEPTH < nch:
            in_copy(c + _DEPTH, slot).start()

    for c in range(nch - _DEPTH, nch):
        out_copy(c, c % _DEPTH).wait()


@functools.partial(jax.jit, static_argnames=("interpret",))
def kernel(x, W1, b1, W2, b2, W3, b3, bn0_mean, bn0_var, bn1_mean, bn1_var,
           bn2_mean, bn2_var, interpret=False):
    n = x.shape[0]
    any_spec = pl.BlockSpec(memory_space=pl.MemorySpace.ANY)
    vmem = pl.BlockSpec(memory_space=pltpu.MemorySpace.VMEM)

    return pl.pallas_call(
        _mlp_pipeline,
        in_specs=[any_spec] + [vmem] * 12,
        out_specs=any_spec,
        out_shape=jax.ShapeDtypeStruct((n, _D_OUT), jnp.float32),
        scratch_shapes=[
            pltpu.VMEM((_DEPTH, _CHUNK, _D_IN), jnp.float32),
            pltpu.VMEM((_DEPTH, _CHUNK, _D_OUT), jnp.float32),
            pltpu.SemaphoreType.DMA((_DEPTH,)),
            pltpu.SemaphoreType.DMA((_DEPTH,)),
            pltpu.VMEM((_H1, _D_IN), jnp.bfloat16),
            pltpu.VMEM((1, _H1), jnp.float32),
            pltpu.VMEM((_H2, _H1), jnp.bfloat16),
            pltpu.VMEM((1, _H2), jnp.float32),
            pltpu.VMEM((_D_OUT, _H2), jnp.bfloat16),
            pltpu.VMEM((1, _D_OUT), jnp.float32),
        ],
        interpret=interpret,
    )(x, W1, b1.reshape(1, -1), W2, b2.reshape(1, -1), W3,
      b3.reshape(1, -1), bn0_mean.reshape(1, -1), bn0_var.reshape(1, -1),
      bn1_mean.reshape(1, -1), bn1_var.reshape(1, -1),
      bn2_mean.reshape(1, -1), bn2_var.reshape(1, -1))


# speedup vs baseline: 1.0038x; 1.0038x over previous
import functools

import jax
import jax.numpy as jnp
from jax.experimental import pallas as pl
from jax.experimental.pallas import tpu as pltpu

_N = 8192
_D_IN = 2048
_D_OUT = 2048
_H1 = 100
_H2 = 50
_EPS = 1e-5
_CHUNK = 512
_IDEPTH = 8
_ODEPTH = 4


def _rm_dot(a, b):
    return jax.lax.dot_general(
        a, b, (((1,), (1,)), ((), ())),
        preferred_element_type=jnp.float32)


def _mlp_pipeline(x_hbm, w1_ref, b1_ref, w2_ref, b2_ref, w3_ref, b3_ref,
                  m0_ref, v0_ref, m1_ref, v1_ref, m2_ref, v2_ref, out_hbm,
                  xbuf, obuf, insems, outsems, w1s, b1s, w2s, b2s, w3s, b3s):
    nch = _N // _CHUNK

    def in_copy(c):
        slot = c % _IDEPTH
        return pltpu.make_async_copy(
            x_hbm.at[pl.ds(c * _CHUNK, _CHUNK), :], xbuf.at[slot],
            insems.at[slot])

    def out_copy(c):
        slot = c % _ODEPTH
        return pltpu.make_async_copy(
            obuf.at[slot], out_hbm.at[pl.ds(c * _CHUNK, _CHUNK), :],
            outsems.at[slot])

    for s in range(_IDEPTH):
        in_copy(s).start()

    s0 = jax.lax.rsqrt(v0_ref[...] + _EPS)
    w1s[...] = w1_ref[...] * s0
    b1s[...] = b1_ref[...] - _rm_dot(m0_ref[...] * s0, w1_ref[...])
    s1 = jax.lax.rsqrt(v1_ref[...] + _EPS)
    s2 = jax.lax.rsqrt(v2_ref[...] + _EPS)
    w2s[...] = w2_ref[...] * s1
    b2s[...] = b2_ref[...] - _rm_dot(m1_ref[...] * s1, w2_ref[...])
    w3s[...] = w3_ref[...] * s2
    b3s[...] = b3_ref[...] - _rm_dot(m2_ref[...] * s2, w3_ref[...])

    for c in range(nch):
        in_copy(c).wait()
        if c >= _ODEPTH:
            out_copy(c - _ODEPTH).wait()
        h = jnp.maximum(_rm_dot(xbuf[c % _IDEPTH], w1s[...]) + b1s[...], 0.0)
        g = jnp.maximum(_rm_dot(h, w2s[...]) + b2s[...], 0.0)
        obuf[c % _ODEPTH] = _rm_dot(g, w3s[...]) + b3s[...]
        out_copy(c).start()
        if c + _IDEPTH < nch:
            in_copy(c + _IDEPTH).start()

    for c in range(nch - _ODEPTH, nch):
        out_copy(c).wait()


@functools.partial(jax.jit, static_argnames=("interpret",))
def kernel(x, W1, b1, W2, b2, W3, b3, bn0_mean, bn0_var, bn1_mean, bn1_var,
           bn2_mean, bn2_var, interpret=False):
    n = x.shape[0]
    any_spec = pl.BlockSpec(memory_space=pl.MemorySpace.ANY)
    vmem = pl.BlockSpec(memory_space=pltpu.MemorySpace.VMEM)

    return pl.pallas_call(
        _mlp_pipeline,
        in_specs=[any_spec] + [vmem] * 12,
        out_specs=any_spec,
        out_shape=jax.ShapeDtypeStruct((n, _D_OUT), jnp.float32),
        scratch_shapes=[
            pltpu.VMEM((_IDEPTH, _CHUNK, _D_IN), jnp.float32),
            pltpu.VMEM((_ODEPTH, _CHUNK, _D_OUT), jnp.float32),
            pltpu.SemaphoreType.DMA((_IDEPTH,)),
            pltpu.SemaphoreType.DMA((_ODEPTH,)),
            pltpu.VMEM((_H1, _D_IN), jnp.float32),
            pltpu.VMEM((1, _H1), jnp.float32),
            pltpu.VMEM((_H2, _H1), jnp.float32),
            pltpu.VMEM((1, _H2), jnp.float32),
            pltpu.VMEM((_D_OUT, _H2), jnp.float32),
            pltpu.VMEM((1, _D_OUT), jnp.float32),
        ],
        interpret=interpret,
    )(x, W1, b1.reshape(1, -1), W2, b2.reshape(1, -1), W3,
      b3.reshape(1, -1), bn0_mean.reshape(1, -1), bn0_var.reshape(1, -1),
      bn1_mean.reshape(1, -1), bn1_var.reshape(1, -1),
      bn2_mean.reshape(1, -1), bn2_var.reshape(1, -1))


# speedup vs baseline: 1.0063x; 1.0025x over previous
import functools

import jax
import jax.numpy as jnp
from jax.experimental import pallas as pl
from jax.experimental.pallas import tpu as pltpu

_N = 8192
_D_IN = 2048
_D_OUT = 2048
_H1 = 100
_H2 = 50
_EPS = 1e-5
_CHUNK = 512
_DEPTH = 6

_SIZES = [128, 128, 256] + [512] * 14 + [256, 128, 128]
_SCHED = []
_off = 0
for _sz in _SIZES:
    _SCHED.append((_off, _sz))
    _off += _sz
assert _off == _N


def _rm_dot(a, b):
    return jax.lax.dot_general(
        a, b, (((1,), (1,)), ((), ())),
        preferred_element_type=jnp.float32)


def _mlp_pipeline(x_hbm, w1_ref, b1_ref, w2_ref, b2_ref, w3_ref, b3_ref,
                  m0_ref, v0_ref, m1_ref, v1_ref, m2_ref, v2_ref, out_hbm,
                  xbuf, obuf, insems, outsems, w1s, b1s, w2s, b2s, w3s, b3s):
    nch = len(_SCHED)

    def in_copy(c, slot):
        off, size = _SCHED[c]
        return pltpu.make_async_copy(
            x_hbm.at[pl.ds(off, size), :], xbuf.at[slot, pl.ds(0, size), :],
            insems.at[slot])

    def out_copy(c, slot):
        off, size = _SCHED[c]
        return pltpu.make_async_copy(
            obuf.at[slot, pl.ds(0, size), :], out_hbm.at[pl.ds(off, size), :],
            outsems.at[slot])

    for s in range(_DEPTH):
        in_copy(s, s).start()

    s0 = jax.lax.rsqrt(v0_ref[...] + _EPS)
    w1s[...] = w1_ref[...] * s0
    b1s[...] = b1_ref[...] - _rm_dot(m0_ref[...] * s0, w1_ref[...])
    s1 = jax.lax.rsqrt(v1_ref[...] + _EPS)
    s2 = jax.lax.rsqrt(v2_ref[...] + _EPS)
    w2s[...] = w2_ref[...] * s1
    b2s[...] = b2_ref[...] - _rm_dot(m1_ref[...] * s1, w2_ref[...])
    w3s[...] = w3_ref[...] * s2
    b3s[...] = b3_ref[...] - _rm_dot(m2_ref[...] * s2, w3_ref[...])

    for c in range(nch):
        slot = c % _DEPTH
        size = _SCHED[c][1]
        in_copy(c, slot).wait()
        if c >= _DEPTH:
            out_copy(c - _DEPTH, slot).wait()
        xb = xbuf[slot, pl.ds(0, size), :]
        h = jnp.maximum(_rm_dot(xb, w1s[...]) + b1s[...], 0.0)
        g = jnp.maximum(_rm_dot(h, w2s[...]) + b2s[...], 0.0)
        obuf[slot, pl.ds(0, size), :] = _rm_dot(g, w3s[...]) + b3s[...]
        out_copy(c, slot).start()
        if c + _DEPTH < nch:
            in_copy(c + _DEPTH, slot).start()

    for c in range(nch - _DEPTH, nch):
        out_copy(c, c % _DEPTH).wait()


@functools.partial(jax.jit, static_argnames=("interpret",))
def kernel(x, W1, b1, W2, b2, W3, b3, bn0_mean, bn0_var, bn1_mean, bn1_var,
           bn2_mean, bn2_var, interpret=False):
    n = x.shape[0]
    any_spec = pl.BlockSpec(memory_space=pl.MemorySpace.ANY)
    vmem = pl.BlockSpec(memory_space=pltpu.MemorySpace.VMEM)

    return pl.pallas_call(
        _mlp_pipeline,
        in_specs=[any_spec] + [vmem] * 12,
        out_specs=any_spec,
        out_shape=jax.ShapeDtypeStruct((n, _D_OUT), jnp.float32),
        scratch_shapes=[
            pltpu.VMEM((_DEPTH, _CHUNK, _D_IN), jnp.float32),
            pltpu.VMEM((_DEPTH, _CHUNK, _D_OUT), jnp.float32),
            pltpu.SemaphoreType.DMA((_DEPTH,)),
            pltpu.SemaphoreType.DMA((_DEPTH,)),
            pltpu.VMEM((_H1, _D_IN), jnp.float32),
            pltpu.VMEM((1, _H1), jnp.float32),
            pltpu.VMEM((_H2, _H1), jnp.float32),
            pltpu.VMEM((1, _H2), jnp.float32),
            pltpu.VMEM((_D_OUT, _H2), jnp.float32),
            pltpu.VMEM((1, _D_OUT), jnp.float32),
        ],
        interpret=interpret,
    )(x, W1, b1.reshape(1, -1), W2, b2.reshape(1, -1), W3,
      b3.reshape(1, -1), bn0_mean.reshape(1, -1), bn0_var.reshape(1, -1),
      bn1_mean.reshape(1, -1), bn1_var.reshape(1, -1),
      bn2_mean.reshape(1, -1), bn2_var.reshape(1, -1))
